# baseline (speedup 1.0000x reference)
import functools

import jax
import jax.numpy as jnp
from jax import lax
from jax.experimental import pallas as pl
from jax.experimental.pallas import tpu as pltpu
from jax.experimental.pallas import tpu_sc as plsc

_N = 10000
_E = 320000
_G = 128
_NP = 10240
_K = 128
_C = 160
_GRP = 5
_GC = _C // _GRP
_EPS = _C * _K
_EPAD = 16 * _EPS
_EW = _EPAD // 32
_BN = 256
_NB = _NP // _BN

_MESH = plsc.VectorSubcoreMesh(core_axis_name="c", subcore_axis_name="s")


@functools.partial(
    pl.kernel,
    out_type=jax.ShapeDtypeStruct((32, _NP), jnp.float32),
    mesh=_MESH,
    scratch_types=[
        pltpu.VMEM((_EW,), jnp.int32),
        pltpu.VMEM((_NP,), jnp.float32),
    ],
    compiler_params=pltpu.CompilerParams(needs_layout_passes=False,
                                         disable_bounds_checks=True),
)
def _deg_kernel(dst_hbm, out_hbm, idx_v, deg_v):
    c = lax.axis_index("c")
    s = lax.axis_index("s")
    w = s * 2 + c

    def zero_body(j, carry):
        deg_v[pl.ds(j * 16, 16)] = jnp.zeros((16,), jnp.float32)
        return carry

    lax.fori_loop(0, _NP // 16, zero_body, 0)

    pltpu.sync_copy(dst_hbm.at[pl.ds(w * _EW, _EW)], idx_v)

    ones = jnp.ones((16,), jnp.float32)

    def body(j, carry):
        idx = idx_v[pl.ds(j * 16, 16)]
        plsc.addupdate_scatter(deg_v, [idx], ones)
        return carry

    lax.fori_loop(0, _EW // 16, body, 0)
    pltpu.sync_copy(deg_v, out_hbm.at[w])


@functools.partial(
    pl.kernel,
    out_type=jax.ShapeDtypeStruct((2 * _NP, 128), jnp.float32),
    mesh=_MESH,
    scratch_types=[
        pltpu.VMEM_SHARED((_NP, 128), jnp.float32),
        pltpu.VMEM((2 * _GC, _K), jnp.int32),
        pltpu.VMEM((_K,), jnp.int32),
        pltpu.VMEM((_K,), jnp.int32),
        pltpu.VMEM((_K, 128), jnp.float32),
        pltpu.SemaphoreType.DMA,
    ],
    compiler_params=pltpu.CompilerParams(disable_bounds_checks=True),
)
def _scatter_kernel(table_hbm, sd_hbm, out_hbm,
                    acc, sd_big, src_v, dst_v, rows, sem):
    c = lax.axis_index("c")
    s = lax.axis_index("s")
    nrow = _NP // 16

    def init_body(t, carry):
        pltpu.sync_copy(table_hbm.at[pl.ds(c * _NP + s * nrow + t * _K, _K)],
                        rows)
        pltpu.sync_copy(rows, acc.at[pl.ds(s * nrow + t * _K, _K)])
        return carry

    lax.fori_loop(0, nrow // _K, init_body, 0)

    plsc.subcore_barrier()

    base = (c * 16 + s) * _GRP

    def group_body(g, carry):
        pltpu.sync_copy(sd_hbm.at[base + g], sd_big)

        def chunk_body(k, carry2):
            for j in range(_K // 16):
                sl = pl.ds(j * 16, 16)
                src_v[sl] = sd_big[2 * k, sl]
                dst_v[sl] = sd_big[2 * k + 1, sl]
            pltpu.async_copy(table_hbm.at[src_v], rows, sem).wait()
            pltpu.sync_copy(rows, acc.at[dst_v], add=True)
            return carry2

        lax.fori_loop(0, _GC, chunk_body, 0)
        return carry

    lax.fori_loop(0, _GRP, group_body, 0)

    plsc.subcore_barrier()

    def out_body(t, carry):
        pltpu.sync_copy(acc.at[pl.ds(s * nrow + t * _K, _K)], rows)
        pltpu.sync_copy(rows,
                        out_hbm.at[pl.ds(c * _NP + s * nrow + t * _K, _K)])
        return carry

    lax.fori_loop(0, nrow // _K, out_body, 0)


def _mm1_body(x_ref, w_ref, degp_ref, hs_ref, dinv_ref):
    deg = jnp.sum(degp_ref[...], axis=0) + 1.0
    dinv = lax.rsqrt(jnp.maximum(deg, 1.0))
    dinv_ref[...] = dinv
    m = jnp.dot(x_ref[...], w_ref[...], preferred_element_type=jnp.float32)
    hs_ref[...] = m * dinv[:, None]


def _mm1(xp, W1, degp):
    return pl.pallas_call(
        _mm1_body,
        grid=(2, _NB),
        in_specs=[
            pl.BlockSpec((_BN, 128), lambda h, i: (i, 0)),
            pl.BlockSpec((128, 128), lambda h, i: (0, h)),
            pl.BlockSpec((32, _BN), lambda h, i: (0, i)),
        ],
        out_specs=[
            pl.BlockSpec((_BN, 128), lambda h, i: (h * _NB + i, 0)),
            pl.BlockSpec((_BN,), lambda h, i: (i,)),
        ],
        out_shape=[
            jax.ShapeDtypeStruct((2 * _NP, 128), jnp.float32),
            jax.ShapeDtypeStruct((_NP,), jnp.float32),
        ],
    )(xp, W1, degp)


def _mm2_body(a_ref, b_ref, dinv_ref, b1_ref, w2_ref, out_ref):
    dinv = dinv_ref[...]
    h = jnp.concatenate([a_ref[...], b_ref[...]], axis=1)
    h = jnp.maximum(h * dinv[:, None] + b1_ref[...][None, :], 0.0)
    m = jnp.dot(h, w2_ref[...], preferred_element_type=jnp.float32)
    out_ref[...] = m * dinv[:, None]


def _mm2(agg1, dinv, b1, W2):
    return pl.pallas_call(
        _mm2_body,
        grid=(2, _NB),
        in_specs=[
            pl.BlockSpec((_BN, 128), lambda h, i: (i, 0)),
            pl.BlockSpec((_BN, 128), lambda h, i: (_NB + i, 0)),
            pl.BlockSpec((_BN,), lambda h, i: (i,)),
            pl.BlockSpec((256,), lambda h, i: (0,)),
            pl.BlockSpec((256, 128), lambda h, i: (0, h)),
        ],
        out_specs=pl.BlockSpec((_BN, 128), lambda h, i: (h * _NB + i, 0)),
        out_shape=jax.ShapeDtypeStruct((2 * _NP, 128), jnp.float32),
    )(agg1, agg1, dinv, b1, W2)


def _pool_body(a_ref, b_ref, dinv_ref, b2_ref, batch_ref, wh_ref, bh_ref,
               out_ref, pool, counts):
    i = pl.program_id(0)

    @pl.when(i == 0)
    def _():
        pool[...] = jnp.zeros_like(pool)
        counts[...] = jnp.zeros_like(counts)

    dinv = dinv_ref[...]
    h = jnp.concatenate([a_ref[...], b_ref[...]], axis=1)
    h = jnp.maximum(h * dinv[:, None] + b2_ref[...][None, :], 0.0)
    bids = batch_ref[...]
    gids = lax.broadcasted_iota(jnp.int32, (_G, _BN), 0)
    mask = (gids == bids[None, :]).astype(jnp.float32)
    pool[...] += jnp.dot(mask, h, preferred_element_type=jnp.float32)
    counts[...] += jnp.sum(mask, axis=1, keepdims=True)

    @pl.when(i == _NB - 1)
    def _():
        g = pool[...] / jnp.maximum(counts[...], 1.0)
        out_ref[...] = (jnp.dot(g, wh_ref[...],
                                preferred_element_type=jnp.float32)
                        + bh_ref[...][None, :])


def _pool(agg2, dinv, b2, batchp, Wh, bh):
    return pl.pallas_call(
        _pool_body,
        grid=(_NB,),
        in_specs=[
            pl.BlockSpec((_BN, 128), lambda i: (i, 0)),
            pl.BlockSpec((_BN, 128), lambda i: (_NB + i, 0)),
            pl.BlockSpec((_BN,), lambda i: (i,)),
            pl.BlockSpec((256,), lambda i: (0,)),
            pl.BlockSpec((_BN,), lambda i: (i,)),
            pl.BlockSpec((256, 10), lambda i: (0, 0)),
            pl.BlockSpec((10,), lambda i: (0,)),
        ],
        out_specs=pl.BlockSpec((_G, 10), lambda i: (0, 0)),
        out_shape=jax.ShapeDtypeStruct((_G, 10), jnp.float32),
        scratch_shapes=[
            pltpu.VMEM((_G, 256), jnp.float32),
            pltpu.VMEM((_G, 1), jnp.float32),
        ],
    )(agg2, agg2, dinv, b2, batchp, Wh, bh)


def kernel(x, edge_index, batch, W1, b1, W2, b2, Wh, bh):
    pad = _EPAD - _E
    src = edge_index[0]
    dst = edge_index[1]
    srcp = jnp.concatenate([src, jnp.full((pad,), _NP - 1, jnp.int32)])
    dstp = jnp.concatenate([dst, jnp.full((pad,), _NP - 1, jnp.int32)])
    d_r = dstp.reshape(16, _GRP, _GC, 1, _K)
    sd = jnp.stack([
        jnp.concatenate(
            [(srcp + c * _NP).reshape(16, _GRP, _GC, 1, _K), d_r], axis=3)
        for c in (0, 1)
    ]).reshape(32 * _GRP, 2 * _GC, _K)

    xp = jnp.pad(x, ((0, _NP - _N), (0, 0)))
    batchp = jnp.pad(batch, (0, _NP - _N), constant_values=_G)

    degp = _deg_kernel(dstp)
    hs1, dinv = _mm1(xp, W1, degp)
    agg1 = _scatter_kernel(hs1, sd)
    hs2 = _mm2(agg1, dinv, b1, W2)
    agg2 = _scatter_kernel(hs2, sd)
    return _pool(agg2, dinv, b2, batchp, Wh, bh)


# speedup vs baseline: 1.6429x; 1.6429x over previous
import functools

import jax
import jax.numpy as jnp
from jax import lax
from jax.experimental import pallas as pl
from jax.experimental.pallas import tpu as pltpu
from jax.experimental.pallas import tpu_sc as plsc

_N = 10000
_E = 320000
_G = 128
_NP = 10240
_K = 128
_C = 157
_EPS = _C * _K
_EPAD = 16 * _EPS
_EW = _EPAD // 32
_BN = 256
_NB = _NP // _BN

_MESH = plsc.VectorSubcoreMesh(core_axis_name="c", subcore_axis_name="s")


@functools.partial(
    pl.kernel,
    out_type=jax.ShapeDtypeStruct((32, _NP), jnp.float32),
    mesh=_MESH,
    scratch_types=[
        pltpu.VMEM((_EW,), jnp.int32),
        pltpu.VMEM((_NP,), jnp.float32),
    ],
    compiler_params=pltpu.CompilerParams(needs_layout_passes=False,
                                         disable_bounds_checks=True),
)
def _deg_kernel(dst_hbm, out_hbm, idx_v, deg_v):
    c = lax.axis_index("c")
    s = lax.axis_index("s")
    w = s * 2 + c

    def zero_body(j, carry):
        deg_v[pl.ds(j * 16, 16)] = jnp.zeros((16,), jnp.float32)
        return carry

    lax.fori_loop(0, _NP // 16, zero_body, 0)

    pltpu.sync_copy(dst_hbm.at[pl.ds(w * _EW, _EW)], idx_v)

    ones = jnp.ones((16,), jnp.float32)

    def body(j, carry):
        idx = idx_v[pl.ds(j * 16, 16)]
        plsc.addupdate_scatter(deg_v, [idx], ones)
        return carry

    lax.fori_loop(0, _EW // 16, body, 0)
    pltpu.sync_copy(deg_v, out_hbm.at[w])


@functools.partial(
    pl.kernel,
    out_type=jax.ShapeDtypeStruct((2 * _NP, 128), jnp.float32),
    mesh=_MESH,
    scratch_types=[
        pltpu.VMEM_SHARED((_NP, 128), jnp.float32),
        pltpu.VMEM((_K,), jnp.int32),
        pltpu.VMEM((_K,), jnp.int32),
        pltpu.VMEM((_K, 128), jnp.float32),
        pltpu.SemaphoreType.DMA,
    ],
)
def _scatter_kernel(table_hbm, src_hbm, dst_hbm, out_hbm,
                    acc, src_v, dst_v, rows, sem):
    c = lax.axis_index("c")
    s = lax.axis_index("s")
    nrow = _NP // 16

    def init_body(t, carry):
        pltpu.sync_copy(table_hbm.at[pl.ds(c * _NP + s * nrow + t * _K, _K)],
                        rows)
        pltpu.sync_copy(rows, acc.at[pl.ds(s * nrow + t * _K, _K)])
        return carry

    lax.fori_loop(0, nrow // _K, init_body, 0)

    plsc.subcore_barrier()

    def edge_body(k, carry):
        base = s * _EPS + k * _K
        pltpu.sync_copy(src_hbm.at[pl.ds(c * _EPAD + base, _K)], src_v)
        d = pltpu.async_copy(table_hbm.at[src_v], rows, sem)
        pltpu.sync_copy(dst_hbm.at[pl.ds(base, _K)], dst_v)
        d.wait()
        pltpu.sync_copy(rows, acc.at[dst_v], add=True)
        return carry

    lax.fori_loop(0, _C, edge_body, 0)

    plsc.subcore_barrier()

    def out_body(t, carry):
        pltpu.sync_copy(acc.at[pl.ds(s * nrow + t * _K, _K)], rows)
        pltpu.sync_copy(rows,
                        out_hbm.at[pl.ds(c * _NP + s * nrow + t * _K, _K)])
        return carry

    lax.fori_loop(0, nrow // _K, out_body, 0)


def _mm1_body(x_ref, w_ref, degp_ref, hs_ref, dinv_ref):
    deg = jnp.sum(degp_ref[...], axis=0) + 1.0
    dinv = lax.rsqrt(jnp.maximum(deg, 1.0))
    dinv_ref[...] = dinv
    m = jnp.dot(x_ref[...], w_ref[...], preferred_element_type=jnp.float32)
    hs_ref[...] = m * dinv[:, None]


def _mm1(xp, W1, degp):
    return pl.pallas_call(
        _mm1_body,
        grid=(2, _NB),
        in_specs=[
            pl.BlockSpec((_BN, 128), lambda h, i: (i, 0)),
            pl.BlockSpec((128, 128), lambda h, i: (0, h)),
            pl.BlockSpec((32, _BN), lambda h, i: (0, i)),
        ],
        out_specs=[
            pl.BlockSpec((_BN, 128), lambda h, i: (h * _NB + i, 0)),
            pl.BlockSpec((_BN,), lambda h, i: (i,)),
        ],
        out_shape=[
            jax.ShapeDtypeStruct((2 * _NP, 128), jnp.float32),
            jax.ShapeDtypeStruct((_NP,), jnp.float32),
        ],
    )(xp, W1, degp)


def _mm2_body(a_ref, b_ref, dinv_ref, b1_ref, w2_ref, out_ref):
    dinv = dinv_ref[...]
    h = jnp.concatenate([a_ref[...], b_ref[...]], axis=1)
    h = jnp.maximum(h * dinv[:, None] + b1_ref[...][None, :], 0.0)
    m = jnp.dot(h, w2_ref[...], preferred_element_type=jnp.float32)
    out_ref[...] = m * dinv[:, None]


def _mm2(agg1, dinv, b1, W2):
    return pl.pallas_call(
        _mm2_body,
        grid=(2, _NB),
        in_specs=[
            pl.BlockSpec((_BN, 128), lambda h, i: (i, 0)),
            pl.BlockSpec((_BN, 128), lambda h, i: (_NB + i, 0)),
            pl.BlockSpec((_BN,), lambda h, i: (i,)),
            pl.BlockSpec((256,), lambda h, i: (0,)),
            pl.BlockSpec((256, 128), lambda h, i: (0, h)),
        ],
        out_specs=pl.BlockSpec((_BN, 128), lambda h, i: (h * _NB + i, 0)),
        out_shape=jax.ShapeDtypeStruct((2 * _NP, 128), jnp.float32),
    )(agg1, agg1, dinv, b1, W2)


def _pool_body(a_ref, b_ref, dinv_ref, b2_ref, batch_ref, wh_ref, bh_ref,
               out_ref, pool, counts):
    i = pl.program_id(0)

    @pl.when(i == 0)
    def _():
        pool[...] = jnp.zeros_like(pool)
        counts[...] = jnp.zeros_like(counts)

    dinv = dinv_ref[...]
    h = jnp.concatenate([a_ref[...], b_ref[...]], axis=1)
    h = jnp.maximum(h * dinv[:, None] + b2_ref[...][None, :], 0.0)
    bids = batch_ref[...]
    gids = lax.broadcasted_iota(jnp.int32, (_G, _BN), 0)
    mask = (gids == bids[None, :]).astype(jnp.float32)
    pool[...] += jnp.dot(mask, h, preferred_element_type=jnp.float32)
    counts[...] += jnp.sum(mask, axis=1, keepdims=True)

    @pl.when(i == _NB - 1)
    def _():
        g = pool[...] / jnp.maximum(counts[...], 1.0)
        out_ref[...] = (jnp.dot(g, wh_ref[...],
                                preferred_element_type=jnp.float32)
                        + bh_ref[...][None, :])


def _pool(agg2, dinv, b2, batchp, Wh, bh):
    return pl.pallas_call(
        _pool_body,
        grid=(_NB,),
        in_specs=[
            pl.BlockSpec((_BN, 128), lambda i: (i, 0)),
            pl.BlockSpec((_BN, 128), lambda i: (_NB + i, 0)),
            pl.BlockSpec((_BN,), lambda i: (i,)),
            pl.BlockSpec((256,), lambda i: (0,)),
            pl.BlockSpec((_BN,), lambda i: (i,)),
            pl.BlockSpec((256, 10), lambda i: (0, 0)),
            pl.BlockSpec((10,), lambda i: (0,)),
        ],
        out_specs=pl.BlockSpec((_G, 10), lambda i: (0, 0)),
        out_shape=jax.ShapeDtypeStruct((_G, 10), jnp.float32),
        scratch_shapes=[
            pltpu.VMEM((_G, 256), jnp.float32),
            pltpu.VMEM((_G, 1), jnp.float32),
        ],
    )(agg2, agg2, dinv, b2, batchp, Wh, bh)


def kernel(x, edge_index, batch, W1, b1, W2, b2, Wh, bh):
    pad = _EPAD - _E
    src = edge_index[0]
    dst = edge_index[1]
    srcp = jnp.concatenate([src, jnp.full((pad,), _NP - 1, jnp.int32)])
    dstp = jnp.concatenate([dst, jnp.full((pad,), _NP - 1, jnp.int32)])
    src2 = jnp.concatenate([srcp, srcp + _NP])

    xp = jnp.pad(x, ((0, _NP - _N), (0, 0)))
    batchp = jnp.pad(batch, (0, _NP - _N), constant_values=_G)

    degp = _deg_kernel(dstp)
    hs1, dinv = _mm1(xp, W1, degp)
    agg1 = _scatter_kernel(hs1, src2, dstp)
    hs2 = _mm2(agg1, dinv, b1, W2)
    agg2 = _scatter_kernel(hs2, src2, dstp)
    return _pool(agg2, dinv, b2, batchp, Wh, bh)


# speedup vs baseline: 1.7911x; 1.0902x over previous
import functools

import jax
import jax.numpy as jnp
from jax import lax
from jax.experimental import pallas as pl
from jax.experimental.pallas import tpu as pltpu
from jax.experimental.pallas import tpu_sc as plsc

_N = 10000
_E = 320000
_G = 128
_NP = 10240
_K = 128
_C = 157
_EPS = _C * _K
_EPAD = 16 * _EPS
_EW = _EPAD // 32
_BN = 256
_NB = _NP // _BN

_MESH = plsc.VectorSubcoreMesh(core_axis_name="c", subcore_axis_name="s")


@functools.partial(
    pl.kernel,
    out_type=jax.ShapeDtypeStruct((32, _NP), jnp.float32),
    mesh=_MESH,
    scratch_types=[
        pltpu.VMEM((_EW,), jnp.int32),
        pltpu.VMEM((_NP,), jnp.float32),
    ],
    compiler_params=pltpu.CompilerParams(needs_layout_passes=False,
                                         disable_bounds_checks=True),
)
def _deg_kernel(dst_hbm, out_hbm, idx_v, deg_v):
    c = lax.axis_index("c")
    s = lax.axis_index("s")
    w = s * 2 + c

    def zero_body(j, carry):
        deg_v[pl.ds(j * 16, 16)] = jnp.zeros((16,), jnp.float32)
        return carry

    lax.fori_loop(0, _NP // 16, zero_body, 0)

    pltpu.sync_copy(dst_hbm.at[pl.ds(w * _EW, _EW)], idx_v)

    ones = jnp.ones((16,), jnp.float32)

    def body(j, carry):
        idx = idx_v[pl.ds(j * 16, 16)]
        plsc.addupdate_scatter(deg_v, [idx], ones)
        return carry

    lax.fori_loop(0, _EW // 16, body, 0)
    pltpu.sync_copy(deg_v, out_hbm.at[w])


@functools.partial(
    pl.kernel,
    out_type=jax.ShapeDtypeStruct((2 * _NP, 128), jnp.float32),
    mesh=_MESH,
    scratch_types=[
        pltpu.VMEM_SHARED((_NP, 128), jnp.float32),
        pltpu.VMEM((_K,), jnp.int32),
        pltpu.VMEM((_K,), jnp.int32),
        pltpu.VMEM((_K, 128), jnp.float32),
        pltpu.SemaphoreType.DMA,
        pltpu.SemaphoreType.DMA,
    ],
)
def _scatter_kernel(table_hbm, src_hbm, dst_hbm, out_hbm,
                    acc, src_v, dst_v, rows, sem, sem_i):
    c = lax.axis_index("c")
    s = lax.axis_index("s")
    nrow = _NP // 16

    def init_body(t, carry):
        pltpu.sync_copy(table_hbm.at[pl.ds(c * _NP + s * nrow + t * _K, _K)],
                        rows)
        pltpu.sync_copy(rows, acc.at[pl.ds(s * nrow + t * _K, _K)])
        return carry

    lax.fori_loop(0, nrow // _K, init_body, 0)

    plsc.subcore_barrier()

    pltpu.sync_copy(src_hbm.at[pl.ds(c * _EPAD + s * _EPS, _K)], src_v)

    def edge_body(k, carry):
        base = s * _EPS + k * _K
        d = pltpu.async_copy(table_hbm.at[src_v], rows, sem)
        pltpu.sync_copy(dst_hbm.at[pl.ds(base, _K)], dst_v)
        d.wait()
        d_i = pltpu.async_copy(
            src_hbm.at[pl.ds(c * _EPAD + base + _K, _K)], src_v, sem_i)
        pltpu.sync_copy(rows, acc.at[dst_v], add=True)
        d_i.wait()
        return carry

    lax.fori_loop(0, _C, edge_body, 0)

    plsc.subcore_barrier()

    def out_body(t, carry):
        pltpu.sync_copy(acc.at[pl.ds(s * nrow + t * _K, _K)], rows)
        pltpu.sync_copy(rows,
                        out_hbm.at[pl.ds(c * _NP + s * nrow + t * _K, _K)])
        return carry

    lax.fori_loop(0, nrow // _K, out_body, 0)


def _mm1_body(x_ref, w_ref, degp_ref, hs_ref, dinv_ref):
    deg = jnp.sum(degp_ref[...], axis=0) + 1.0
    dinv = lax.rsqrt(jnp.maximum(deg, 1.0))
    dinv_ref[...] = dinv
    m = jnp.dot(x_ref[...], w_ref[...], preferred_element_type=jnp.float32)
    hs_ref[...] = m * dinv[:, None]


def _mm1(xp, W1, degp):
    return pl.pallas_call(
        _mm1_body,
        grid=(2, _NB),
        in_specs=[
            pl.BlockSpec((_BN, 128), lambda h, i: (i, 0)),
            pl.BlockSpec((128, 128), lambda h, i: (0, h)),
            pl.BlockSpec((32, _BN), lambda h, i: (0, i)),
        ],
        out_specs=[
            pl.BlockSpec((_BN, 128), lambda h, i: (h * _NB + i, 0)),
            pl.BlockSpec((_BN,), lambda h, i: (i,)),
        ],
        out_shape=[
            jax.ShapeDtypeStruct((2 * _NP, 128), jnp.float32),
            jax.ShapeDtypeStruct((_NP,), jnp.float32),
        ],
    )(xp, W1, degp)


def _mm2_body(a_ref, b_ref, dinv_ref, b1_ref, w2_ref, out_ref):
    dinv = dinv_ref[...]
    h = jnp.concatenate([a_ref[...], b_ref[...]], axis=1)
    h = jnp.maximum(h * dinv[:, None] + b1_ref[...][None, :], 0.0)
    m = jnp.dot(h, w2_ref[...], preferred_element_type=jnp.float32)
    out_ref[...] = m * dinv[:, None]


def _mm2(agg1, dinv, b1, W2):
    return pl.pallas_call(
        _mm2_body,
        grid=(2, _NB),
        in_specs=[
            pl.BlockSpec((_BN, 128), lambda h, i: (i, 0)),
            pl.BlockSpec((_BN, 128), lambda h, i: (_NB + i, 0)),
            pl.BlockSpec((_BN,), lambda h, i: (i,)),
            pl.BlockSpec((256,), lambda h, i: (0,)),
            pl.BlockSpec((256, 128), lambda h, i: (0, h)),
        ],
        out_specs=pl.BlockSpec((_BN, 128), lambda h, i: (h * _NB + i, 0)),
        out_shape=jax.ShapeDtypeStruct((2 * _NP, 128), jnp.float32),
    )(agg1, agg1, dinv, b1, W2)


def _pool_body(a_ref, b_ref, dinv_ref, b2_ref, batch_ref, wh_ref, bh_ref,
               out_ref, pool, counts):
    i = pl.program_id(0)

    @pl.when(i == 0)
    def _():
        pool[...] = jnp.zeros_like(pool)
        counts[...] = jnp.zeros_like(counts)

    dinv = dinv_ref[...]
    h = jnp.concatenate([a_ref[...], b_ref[...]], axis=1)
    h = jnp.maximum(h * dinv[:, None] + b2_ref[...][None, :], 0.0)
    bids = batch_ref[...]
    gids = lax.broadcasted_iota(jnp.int32, (_G, _BN), 0)
    mask = (gids == bids[None, :]).astype(jnp.float32)
    pool[...] += jnp.dot(mask, h, preferred_element_type=jnp.float32)
    counts[...] += jnp.sum(mask, axis=1, keepdims=True)

    @pl.when(i == _NB - 1)
    def _():
        g = pool[...] / jnp.maximum(counts[...], 1.0)
        out_ref[...] = (jnp.dot(g, wh_ref[...],
                                preferred_element_type=jnp.float32)
                        + bh_ref[...][None, :])


def _pool(agg2, dinv, b2, batchp, Wh, bh):
    return pl.pallas_call(
        _pool_body,
        grid=(_NB,),
        in_specs=[
            pl.BlockSpec((_BN, 128), lambda i: (i, 0)),
            pl.BlockSpec((_BN, 128), lambda i: (_NB + i, 0)),
            pl.BlockSpec((_BN,), lambda i: (i,)),
            pl.BlockSpec((256,), lambda i: (0,)),
            pl.BlockSpec((_BN,), lambda i: (i,)),
            pl.BlockSpec((256, 10), lambda i: (0, 0)),
            pl.BlockSpec((10,), lambda i: (0,)),
        ],
        out_specs=pl.BlockSpec((_G, 10), lambda i: (0, 0)),
        out_shape=jax.ShapeDtypeStruct((_G, 10), jnp.float32),
        scratch_shapes=[
            pltpu.VMEM((_G, 256), jnp.float32),
            pltpu.VMEM((_G, 1), jnp.float32),
        ],
    )(agg2, agg2, dinv, b2, batchp, Wh, bh)


def kernel(x, edge_index, batch, W1, b1, W2, b2, Wh, bh):
    pad = _EPAD - _E
    src = edge_index[0]
    dst = edge_index[1]
    srcp = jnp.concatenate([src, jnp.full((pad,), _NP - 1, jnp.int32)])
    dstp = jnp.concatenate([dst, jnp.full((pad,), _NP - 1, jnp.int32)])
    src2 = jnp.concatenate([srcp, srcp + _NP,
                            jnp.zeros((_K,), jnp.int32)])

    xp = jnp.pad(x, ((0, _NP - _N), (0, 0)))
    batchp = jnp.pad(batch, (0, _NP - _N), constant_values=_G)

    degp = _deg_kernel(dstp)
    hs1, dinv = _mm1(xp, W1, degp)
    agg1 = _scatter_kernel(hs1, src2, dstp)
    hs2 = _mm2(agg1, dinv, b1, W2)
    agg2 = _scatter_kernel(hs2, src2, dstp)
    return _pool(agg2, dinv, b2, batchp, Wh, bh)


# speedup vs baseline: 2.1747x; 1.2141x over previous
import functools

import jax
import jax.numpy as jnp
from jax import lax
from jax.experimental import pallas as pl
from jax.experimental.pallas import tpu as pltpu
from jax.experimental.pallas import tpu_sc as plsc

_N = 10000
_E = 320000
_G = 128
_NP = 10240
_K = 128
_C = 157
_EPS = _C * _K
_EPAD = 16 * _EPS
_EW = _EPAD // 32
_BN = 256
_NB = _NP // _BN

_MESH = plsc.VectorSubcoreMesh(core_axis_name="c", subcore_axis_name="s")


@functools.partial(
    pl.kernel,
    out_type=jax.ShapeDtypeStruct((32, _NP), jnp.float32),
    mesh=_MESH,
    scratch_types=[
        pltpu.VMEM((_EW,), jnp.int32),
        pltpu.VMEM((_NP,), jnp.float32),
    ],
    compiler_params=pltpu.CompilerParams(needs_layout_passes=False,
                                         disable_bounds_checks=True),
)
def _deg_kernel(dst_hbm, out_hbm, idx_v, deg_v):
    c = lax.axis_index("c")
    s = lax.axis_index("s")
    w = s * 2 + c

    def zero_body(j, carry):
        deg_v[pl.ds(j * 16, 16)] = jnp.zeros((16,), jnp.float32)
        return carry

    lax.fori_loop(0, _NP // 16, zero_body, 0)

    pltpu.sync_copy(dst_hbm.at[pl.ds(w * _EW, _EW)], idx_v)

    ones = jnp.ones((16,), jnp.float32)

    def body(j, carry):
        idx = idx_v[pl.ds(j * 16, 16)]
        plsc.addupdate_scatter(deg_v, [idx], ones)
        return carry

    lax.fori_loop(0, _EW // 16, body, 0)
    pltpu.sync_copy(deg_v, out_hbm.at[w])


@functools.partial(
    pl.kernel,
    out_type=jax.ShapeDtypeStruct((2 * _NP, 128), jnp.float32),
    mesh=_MESH,
    scratch_types=[
        pltpu.VMEM_SHARED((_NP, 128), jnp.float32),
        pltpu.VMEM((_K,), jnp.int32),
        pltpu.VMEM((_K,), jnp.int32),
        pltpu.VMEM((_K,), jnp.int32),
        pltpu.VMEM((_K,), jnp.int32),
        pltpu.VMEM((_K, 128), jnp.float32),
        pltpu.VMEM((_K, 128), jnp.float32),
        pltpu.SemaphoreType.DMA,
        pltpu.SemaphoreType.DMA,
        pltpu.SemaphoreType.DMA,
        pltpu.SemaphoreType.DMA,
    ],
)
def _scatter_kernel(table_hbm, src_hbm, dst_hbm, out_hbm,
                    acc, src_a, src_b, dst_a, dst_b, rows_a, rows_b,
                    sem_a, sem_b, sem_ia, sem_ib):
    c = lax.axis_index("c")
    s = lax.axis_index("s")
    nrow = _NP // 16

    def init_body(t, carry):
        pltpu.sync_copy(table_hbm.at[pl.ds(c * _NP + s * nrow + t * _K, _K)],
                        rows_a)
        pltpu.sync_copy(rows_a, acc.at[pl.ds(s * nrow + t * _K, _K)])
        return carry

    lax.fori_loop(0, nrow // _K, init_body, 0)

    plsc.subcore_barrier()

    ebase = s * _EPS

    def chunk_off(k):
        return pl.ds(c * _EPAD + ebase + k * _K, _K)

    pltpu.sync_copy(src_hbm.at[chunk_off(0)], src_a)
    g_a = pltpu.async_copy(table_hbm.at[src_a], rows_a, sem_a)
    pltpu.sync_copy(src_hbm.at[chunk_off(1)], src_b)

    def pair_body(p, carry):
        a = 2 * p
        pltpu.sync_copy(dst_hbm.at[pl.ds(ebase + a * _K, _K)], dst_a)
        pltpu.make_async_copy(table_hbm.at[src_a], rows_a, sem_a).wait()
        g_b = pltpu.async_copy(table_hbm.at[src_b], rows_b, sem_b)
        d_ia = pltpu.async_copy(src_hbm.at[chunk_off(a + 2)], src_a, sem_ia)
        pltpu.sync_copy(rows_a, acc.at[dst_a], add=True)
        pltpu.sync_copy(dst_hbm.at[pl.ds(ebase + (a + 1) * _K, _K)], dst_b)
        g_b.wait()
        d_ia.wait()
        pltpu.async_copy(table_hbm.at[src_a], rows_a, sem_a)
        d_ib = pltpu.async_copy(src_hbm.at[chunk_off(a + 3)], src_b, sem_ib)
        pltpu.sync_copy(rows_b, acc.at[dst_b], add=True)
        d_ib.wait()
        return carry

    lax.fori_loop(0, _C // 2, pair_body, 0)

    last = _C - 1
    pltpu.sync_copy(dst_hbm.at[pl.ds(ebase + last * _K, _K)], dst_a)
    pltpu.make_async_copy(table_hbm.at[src_a], rows_a, sem_a).wait()
    pltpu.sync_copy(rows_a, acc.at[dst_a], add=True)

    plsc.subcore_barrier()

    def out_body(t, carry):
        pltpu.sync_copy(acc.at[pl.ds(s * nrow + t * _K, _K)], rows_a)
        pltpu.sync_copy(rows_a,
                        out_hbm.at[pl.ds(c * _NP + s * nrow + t * _K, _K)])
        return carry

    lax.fori_loop(0, nrow // _K, out_body, 0)


def _mm1_body(x_ref, w_ref, degp_ref, hs_ref, dinv_ref):
    deg = jnp.sum(degp_ref[...], axis=0) + 1.0
    dinv = lax.rsqrt(jnp.maximum(deg, 1.0))
    dinv_ref[...] = dinv
    m = jnp.dot(x_ref[...], w_ref[...], preferred_element_type=jnp.float32)
    hs_ref[...] = m * dinv[:, None]


def _mm1(xp, W1, degp):
    return pl.pallas_call(
        _mm1_body,
        grid=(2, _NB),
        in_specs=[
            pl.BlockSpec((_BN, 128), lambda h, i: (i, 0)),
            pl.BlockSpec((128, 128), lambda h, i: (0, h)),
            pl.BlockSpec((32, _BN), lambda h, i: (0, i)),
        ],
        out_specs=[
            pl.BlockSpec((_BN, 128), lambda h, i: (h * _NB + i, 0)),
            pl.BlockSpec((_BN,), lambda h, i: (i,)),
        ],
        out_shape=[
            jax.ShapeDtypeStruct((2 * _NP, 128), jnp.float32),
            jax.ShapeDtypeStruct((_NP,), jnp.float32),
        ],
    )(xp, W1, degp)


def _mm2_body(a_ref, b_ref, dinv_ref, b1_ref, w2_ref, out_ref):
    dinv = dinv_ref[...]
    h = jnp.concatenate([a_ref[...], b_ref[...]], axis=1)
    h = jnp.maximum(h * dinv[:, None] + b1_ref[...][None, :], 0.0)
    m = jnp.dot(h, w2_ref[...], preferred_element_type=jnp.float32)
    out_ref[...] = m * dinv[:, None]


def _mm2(agg1, dinv, b1, W2):
    return pl.pallas_call(
        _mm2_body,
        grid=(2, _NB),
        in_specs=[
            pl.BlockSpec((_BN, 128), lambda h, i: (i, 0)),
            pl.BlockSpec((_BN, 128), lambda h, i: (_NB + i, 0)),
            pl.BlockSpec((_BN,), lambda h, i: (i,)),
            pl.BlockSpec((256,), lambda h, i: (0,)),
            pl.BlockSpec((256, 128), lambda h, i: (0, h)),
        ],
        out_specs=pl.BlockSpec((_BN, 128), lambda h, i: (h * _NB + i, 0)),
        out_shape=jax.ShapeDtypeStruct((2 * _NP, 128), jnp.float32),
    )(agg1, agg1, dinv, b1, W2)


def _pool_body(a_ref, b_ref, dinv_ref, b2_ref, batch_ref, wh_ref, bh_ref,
               out_ref, pool, counts):
    i = pl.program_id(0)

    @pl.when(i == 0)
    def _():
        pool[...] = jnp.zeros_like(pool)
        counts[...] = jnp.zeros_like(counts)

    dinv = dinv_ref[...]
    h = jnp.concatenate([a_ref[...], b_ref[...]], axis=1)
    h = jnp.maximum(h * dinv[:, None] + b2_ref[...][None, :], 0.0)
    bids = batch_ref[...]
    gids = lax.broadcasted_iota(jnp.int32, (_G, _BN), 0)
    mask = (gids == bids[None, :]).astype(jnp.float32)
    pool[...] += jnp.dot(mask, h, preferred_element_type=jnp.float32)
    counts[...] += jnp.sum(mask, axis=1, keepdims=True)

    @pl.when(i == _NB - 1)
    def _():
        g = pool[...] / jnp.maximum(counts[...], 1.0)
        out_ref[...] = (jnp.dot(g, wh_ref[...],
                                preferred_element_type=jnp.float32)
                        + bh_ref[...][None, :])


def _pool(agg2, dinv, b2, batchp, Wh, bh):
    return pl.pallas_call(
        _pool_body,
        grid=(_NB,),
        in_specs=[
            pl.BlockSpec((_BN, 128), lambda i: (i, 0)),
            pl.BlockSpec((_BN, 128), lambda i: (_NB + i, 0)),
            pl.BlockSpec((_BN,), lambda i: (i,)),
            pl.BlockSpec((256,), lambda i: (0,)),
            pl.BlockSpec((_BN,), lambda i: (i,)),
            pl.BlockSpec((256, 10), lambda i: (0, 0)),
            pl.BlockSpec((10,), lambda i: (0,)),
        ],
        out_specs=pl.BlockSpec((_G, 10), lambda i: (0, 0)),
        out_shape=jax.ShapeDtypeStruct((_G, 10), jnp.float32),
        scratch_shapes=[
            pltpu.VMEM((_G, 256), jnp.float32),
            pltpu.VMEM((_G, 1), jnp.float32),
        ],
    )(agg2, agg2, dinv, b2, batchp, Wh, bh)


def kernel(x, edge_index, batch, W1, b1, W2, b2, Wh, bh):
    pad = _EPAD - _E
    src = edge_index[0]
    dst = edge_index[1]
    srcp = jnp.concatenate([src, jnp.full((pad,), _NP - 1, jnp.int32)])
    dstp = jnp.concatenate([dst, jnp.full((pad,), _NP - 1, jnp.int32)])
    src2 = jnp.concatenate([srcp, srcp + _NP,
                            jnp.zeros((_K,), jnp.int32)])

    xp = jnp.pad(x, ((0, _NP - _N), (0, 0)))
    batchp = jnp.pad(batch, (0, _NP - _N), constant_values=_G)

    degp = _deg_kernel(dstp)
    hs1, dinv = _mm1(xp, W1, degp)
    agg1 = _scatter_kernel(hs1, src2, dstp)
    hs2 = _mm2(agg1, dinv, b1, W2)
    agg2 = _scatter_kernel(hs2, src2, dstp)
    return _pool(agg2, dinv, b2, batchp, Wh, bh)
